# baseline (speedup 1.0000x reference)
import functools

import jax
import jax.numpy as jnp
import numpy as np
from jax import lax
from jax.experimental import pallas as pl
from jax.experimental.pallas import tpu as pltpu
from jax.experimental.pallas import tpu_sc as plsc

N = 10000
E = 2000
H = 256
BK = 1000
W = 128
NTILES = 32
EPT = 64
WPT = W // NTILES
EPAD = NTILES * EPT
CAP = EPT * 16
NCHUNK = WPT * N // 16


def _bf(x):
    return x.astype(jnp.bfloat16)


def _prep_kernel(inc_ref, x_ref, w1_ref, incb_ref, xw1_ref):
    incb_ref[...] = _bf(inc_ref[...])
    xw1_ref[...] = jnp.dot(x_ref[...], w1_ref[...],
                           preferred_element_type=jnp.float32)


def _prep(inc, x, w1):
    nk = N // BK
    return pl.pallas_call(
        _prep_kernel,
        grid=(nk,),
        in_specs=[
            pl.BlockSpec((BK, E), lambda k: (k, 0)),
            pl.BlockSpec((BK, H), lambda k: (k, 0)),
            pl.BlockSpec((H, H), lambda k: (0, 0)),
        ],
        out_specs=[
            pl.BlockSpec((BK, E), lambda k: (k, 0)),
            pl.BlockSpec((BK, H), lambda k: (k, 0)),
        ],
        out_shape=[
            jax.ShapeDtypeStruct((N, E), jnp.bfloat16),
            jax.ShapeDtypeStruct((N, H), jnp.float32),
        ],
    )(inc, x, w1)


def _edge_fin_kernel(intra_ref, w2_ref, x1_ref, xw2_ref):
    x1 = jnp.maximum(intra_ref[...], 0.0)
    x1_ref[...] = x1
    xw2_ref[...] = _bf(jnp.dot(x1, w2_ref[...],
                               preferred_element_type=jnp.float32))


def _edge_fin(intra, w2):
    return pl.pallas_call(
        _edge_fin_kernel,
        grid=(1,),
        in_specs=[
            pl.BlockSpec((E, H), lambda k: (0, 0)),
            pl.BlockSpec((H, H), lambda k: (0, 0)),
        ],
        out_specs=[
            pl.BlockSpec((E, H), lambda k: (0, 0)),
            pl.BlockSpec((E, H), lambda k: (0, 0)),
        ],
        out_shape=[
            jax.ShapeDtypeStruct((E, H), jnp.float32),
            jax.ShapeDtypeStruct((E, H), jnp.bfloat16),
        ],
    )(intra, w2)


def _node0_kernel(inc_ref, xw2_ref, w1_ref, xw1_ref):
    t = jnp.maximum(jnp.dot(inc_ref[...], xw2_ref[...],
                            preferred_element_type=jnp.float32), 0.0)
    xw1_ref[...] = jnp.dot(t, w1_ref[...], preferred_element_type=jnp.float32)


def _node0(inc, xw2, w1):
    nm = N // BK
    return pl.pallas_call(
        _node0_kernel,
        grid=(nm,),
        in_specs=[
            pl.BlockSpec((BK, E), lambda m: (m, 0)),
            pl.BlockSpec((E, H), lambda m: (0, 0)),
            pl.BlockSpec((H, H), lambda m: (0, 0)),
        ],
        out_specs=pl.BlockSpec((BK, H), lambda m: (m, 0)),
        out_shape=jax.ShapeDtypeStruct((N, H), jnp.float32),
    )(inc, xw2, w1)


def _node1_kernel(inc_ref, xw2_ref, out_ref):
    out_ref[...] = jnp.maximum(
        jnp.dot(inc_ref[...], xw2_ref[...], preferred_element_type=jnp.float32),
        0.0)


def _node1(inc, xw2):
    nm = N // BK
    return pl.pallas_call(
        _node1_kernel,
        grid=(nm,),
        in_specs=[
            pl.BlockSpec((BK, E), lambda m: (m, 0)),
            pl.BlockSpec((E, H), lambda m: (0, 0)),
        ],
        out_specs=pl.BlockSpec((BK, H), lambda m: (m, 0)),
        out_shape=jax.ShapeDtypeStruct((N, H), jnp.float32),
    )(inc, xw2)


DEG = 16
NPAD = N + 8


def _sc_gather_body(mtab_ref, xw1_ref, out_ref, midx, gbuf, sem):
    wid = lax.axis_index("s") * 2 + lax.axis_index("c")
    pltpu.sync_copy(mtab_ref.at[wid], midx)
    for g in range(EPT * DEG // 128):
        pltpu.async_copy(xw1_ref.at[midx.at[0, pl.ds(g * 128, 128)]],
                         gbuf, sem).wait()
        pltpu.sync_copy(gbuf, out_ref.at[wid, pl.ds(g * 128, 128)])


def _sc_gather(mtab, xw1):
    mesh = plsc.VectorSubcoreMesh(core_axis_name="c", subcore_axis_name="s")
    kern = pl.kernel(
        _sc_gather_body,
        mesh=mesh,
        out_type=jax.ShapeDtypeStruct((NTILES, EPT * DEG, H), jnp.float32),
        scratch_types=[
            pltpu.VMEM((1, EPT * DEG), jnp.int32),
            pltpu.VMEM((128, H), jnp.float32),
            pltpu.SemaphoreType.DMA,
        ],
    )
    return kern(mtab, xw1).reshape(EPAD * DEG, H)


def _seg_matrix():
    m = np.zeros((EPT, EPT * DEG), np.float32)
    j = np.arange(EPT * DEG)
    m[j // DEG, j] = 1.0
    return jnp.asarray(m, dtype=jnp.bfloat16)


def _seg_reduce_kernel(g_ref, s_ref, out_ref):
    out_ref[...] = jnp.dot(s_ref[...], _bf(g_ref[...]),
                           preferred_element_type=jnp.float32)


def _seg_reduce(gath, s64):
    return pl.pallas_call(
        _seg_reduce_kernel,
        grid=(NTILES,),
        in_specs=[
            pl.BlockSpec((EPT * DEG, H), lambda g: (g, 0)),
            pl.BlockSpec((EPT, EPT * DEG), lambda g: (0, 0)),
        ],
        out_specs=pl.BlockSpec((EPT, H), lambda g: (g, 0)),
        out_shape=jax.ShapeDtypeStruct((EPAD, H), jnp.float32),
    )(gath, s64)


def kernel(x_0, incidence_1, weight1_0, weight2_0, att_weight1_0, att_weight2_0,
           weight1_1, weight2_1, att_weight1_1, att_weight2_1):
    vals, idx = lax.top_k(incidence_1.T, DEG)
    mtab = jnp.where(vals > 0.5, idx, N).astype(jnp.int32)
    mtab = jnp.pad(mtab, ((0, EPAD - E), (0, 0)), constant_values=N)
    mtab = mtab.reshape(NTILES, 1, EPT * DEG)

    inc_bf, xw1_0 = _prep(incidence_1, x_0, weight1_0)
    s64 = _seg_matrix()

    xw1_0p = jnp.pad(xw1_0, ((0, NPAD - N), (0, 0)))
    intra_0 = _seg_reduce(_sc_gather(mtab, xw1_0p), s64)
    _, xw2_0 = _edge_fin(intra_0, weight2_0)
    xw1_1 = _node0(inc_bf, xw2_0, weight1_1)

    xw1_1p = jnp.pad(xw1_1, ((0, NPAD - N), (0, 0)))
    intra_1 = _seg_reduce(_sc_gather(mtab, xw1_1p), s64)
    x1_1, xw2_1 = _edge_fin(intra_1, weight2_1)
    x_out = _node1(inc_bf, xw2_1)
    return (x_out, x1_1)


# speedup vs baseline: 15.4832x; 15.4832x over previous
import functools

import jax
import jax.numpy as jnp
from jax import lax
from jax.experimental import pallas as pl
from jax.experimental.pallas import tpu as pltpu

N = 10000
E = 2000
H = 256
BK = 1000


def _bf(x):
    return x.astype(jnp.bfloat16)


def _edge0_kernel(inc_ref, x_ref, w1_ref, w2_ref, x1_ref, xw2_ref, acc_ref,
                  *, nk):
    k = pl.program_id(0)

    @pl.when(k == 0)
    def _init():
        acc_ref[...] = jnp.zeros_like(acc_ref)

    xw1 = jnp.dot(x_ref[...], w1_ref[...], preferred_element_type=jnp.float32)
    acc_ref[...] += jnp.dot(_bf(xw1.T), inc_ref[...],
                            preferred_element_type=jnp.float32)

    @pl.when(k == nk - 1)
    def _fin():
        x1 = jnp.maximum(acc_ref[...].T, 0.0)
        x1_ref[...] = x1
        xw2_ref[...] = _bf(jnp.dot(x1, w2_ref[...],
                                   preferred_element_type=jnp.float32))


def _edge1_kernel(inc_ref, xw1_ref, w2_ref, x1_ref, xw2_ref, acc_ref, *, nk):
    k = pl.program_id(0)

    @pl.when(k == 0)
    def _init():
        acc_ref[...] = jnp.zeros_like(acc_ref)

    acc_ref[...] += jnp.dot(xw1_ref[...].T, inc_ref[...],
                            preferred_element_type=jnp.float32)

    @pl.when(k == nk - 1)
    def _fin():
        x1 = jnp.maximum(acc_ref[...].T, 0.0)
        x1_ref[...] = x1
        xw2_ref[...] = _bf(jnp.dot(x1, w2_ref[...],
                                   preferred_element_type=jnp.float32))


def _node0_kernel(inc_ref, xw2_ref, w1_ref, xw1_ref):
    t = jnp.maximum(jnp.dot(inc_ref[...], xw2_ref[...],
                            preferred_element_type=jnp.float32), 0.0)
    xw1_ref[...] = _bf(jnp.dot(t, w1_ref[...],
                               preferred_element_type=jnp.float32))


def _node1_kernel(inc_ref, xw2_ref, out_ref):
    out_ref[...] = jnp.maximum(
        jnp.dot(inc_ref[...], xw2_ref[...], preferred_element_type=jnp.float32),
        0.0)


def _edge0(inc, x, w1, w2):
    nk = N // BK
    return pl.pallas_call(
        functools.partial(_edge0_kernel, nk=nk),
        grid=(nk,),
        in_specs=[
            pl.BlockSpec((BK, E), lambda k: (k, 0)),
            pl.BlockSpec((BK, H), lambda k: (k, 0)),
            pl.BlockSpec((H, H), lambda k: (0, 0)),
            pl.BlockSpec((H, H), lambda k: (0, 0)),
        ],
        out_specs=[
            pl.BlockSpec((E, H), lambda k: (0, 0)),
            pl.BlockSpec((E, H), lambda k: (0, 0)),
        ],
        out_shape=[
            jax.ShapeDtypeStruct((E, H), jnp.float32),
            jax.ShapeDtypeStruct((E, H), jnp.bfloat16),
        ],
        scratch_shapes=[pltpu.VMEM((H, E), jnp.float32)],
    )(inc, x, w1, w2)


def _edge1(inc, xw1t, w2):
    nk = N // BK
    return pl.pallas_call(
        functools.partial(_edge1_kernel, nk=nk),
        grid=(nk,),
        in_specs=[
            pl.BlockSpec((BK, E), lambda k: (k, 0)),
            pl.BlockSpec((BK, H), lambda k: (k, 0)),
            pl.BlockSpec((H, H), lambda k: (0, 0)),
        ],
        out_specs=[
            pl.BlockSpec((E, H), lambda k: (0, 0)),
            pl.BlockSpec((E, H), lambda k: (0, 0)),
        ],
        out_shape=[
            jax.ShapeDtypeStruct((E, H), jnp.float32),
            jax.ShapeDtypeStruct((E, H), jnp.bfloat16),
        ],
        scratch_shapes=[pltpu.VMEM((H, E), jnp.float32)],
    )(inc, xw1t, w2)


def _node0(inc, xw2, w1):
    nm = N // BK
    return pl.pallas_call(
        _node0_kernel,
        grid=(nm,),
        in_specs=[
            pl.BlockSpec((BK, E), lambda m: (m, 0)),
            pl.BlockSpec((E, H), lambda m: (0, 0)),
            pl.BlockSpec((H, H), lambda m: (0, 0)),
        ],
        out_specs=pl.BlockSpec((BK, H), lambda m: (m, 0)),
        out_shape=jax.ShapeDtypeStruct((N, H), jnp.bfloat16),
    )(inc, xw2, w1)


def _node1(inc, xw2):
    nm = N // BK
    return pl.pallas_call(
        _node1_kernel,
        grid=(nm,),
        in_specs=[
            pl.BlockSpec((BK, E), lambda m: (m, 0)),
            pl.BlockSpec((E, H), lambda m: (0, 0)),
        ],
        out_specs=pl.BlockSpec((BK, H), lambda m: (m, 0)),
        out_shape=jax.ShapeDtypeStruct((N, H), jnp.float32),
    )(inc, xw2)


def kernel(x_0, incidence_1, weight1_0, weight2_0, att_weight1_0, att_weight2_0,
           weight1_1, weight2_1, att_weight1_1, att_weight2_1):
    inc_bf = incidence_1.astype(jnp.bfloat16)
    _, xw2_0 = _edge0(inc_bf, x_0, weight1_0, weight2_0)
    xw1t_1 = _node0(inc_bf, xw2_0, weight1_1)
    x1_1, xw2_1 = _edge1(inc_bf, xw1t_1, weight2_1)
    x_out = _node1(inc_bf, xw2_1)
    return (x_out, x1_1)
